# baseline (speedup 1.0000x reference)
import functools

import jax
import jax.numpy as jnp
from jax import lax
from jax.experimental import pallas as pl
from jax.experimental.pallas import tpu as pltpu
from jax.experimental.pallas import tpu_sc as plsc


def _gather_sc(weights, idxflat, h_dim, d_dim):
    n_rows, lanes = idxflat.shape
    d_hi = d_dim // 8
    b_hi = n_rows // h_dim
    info = plsc.get_sparse_core_info()
    nc, ns = info.num_cores, info.num_subcores
    nw = nc * ns
    rows_per_w = n_rows // nw
    n2 = rows_per_w // 2

    mesh = plsc.VectorSubcoreMesh(core_axis_name="c", subcore_axis_name="s")

    @functools.partial(
        pl.kernel,
        mesh=mesh,
        compiler_params=pltpu.CompilerParams(
            use_tc_tiling_on_sc=False, needs_layout_passes=False
        ),
        out_type=jax.ShapeDtypeStruct((h_dim, d_hi, b_hi, 8, lanes), jnp.float32),
        scratch_types=[
            pltpu.VMEM((rows_per_w, lanes), jnp.int32),
            pltpu.VMEM((lanes, d_dim), jnp.float32),
            pltpu.VMEM((lanes, d_dim), jnp.float32),
            pltpu.VMEM((d_hi, 8, lanes), jnp.float32),
            pltpu.VMEM((d_hi, 8, lanes), jnp.float32),
            pltpu.SemaphoreType.DMA,
            pltpu.SemaphoreType.DMA,
            pltpu.SemaphoreType.DMA,
            pltpu.SemaphoreType.DMA,
        ],
    )
    def k(table_hbm, idx_hbm, out_hbm, idx_all, rows0, rows1, out0, out1,
          sg0, sg1, sw0, sw1):
        wid = lax.axis_index("s") * nc + lax.axis_index("c")
        base = wid * rows_per_w
        rows = (rows0, rows1)
        outs = (out0, out1)
        sg = (sg0, sg1)
        sw = (sw0, sw1)

        pltpu.sync_copy(idx_hbm.at[pl.ds(base, rows_per_w)], idx_all)

        iota = lax.iota(jnp.int32, 16)
        rvecs = [iota + (16 * t) for t in range(lanes // 16)]

        def fire_gather(g, p):
            pltpu.make_async_copy(
                table_hbm.at[idx_all.at[g]], rows[p], sg[p]
            ).start()

        def drain_gather(p):
            pltpu.make_async_copy(
                table_hbm.at[pl.ds(0, lanes)], rows[p], sg[p]
            ).wait()

        def transpose(p):
            r, o = rows[p], outs[p]
            for d in range(d_dim):
                dvec = jnp.full((16,), d, jnp.int32)
                for t in range(lanes // 16):
                    val = plsc.load_gather(r, [rvecs[t], dvec])
                    o[d // 8, d % 8, pl.ds(16 * t, 16)] = val

        def fire_wb(g, p):
            c = base + g
            h = c // b_hi
            bc = lax.rem(c, b_hi)
            for dr in range(d_hi):
                pltpu.make_async_copy(
                    outs[p].at[dr], out_hbm.at[h, dr, bc], sw[p]
                ).start()

        def wait_wb(p):
            for dr in range(d_hi):
                pltpu.make_async_copy(
                    outs[p].at[dr], out_hbm.at[0, dr, 0], sw[p]
                ).wait()

        fire_gather(0, 0)
        fire_gather(1, 1)

        def body(i, carry):
            g0 = 2 * i
            drain_gather(0)

            @pl.when(i > 0)
            def _():
                wait_wb(0)

            transpose(0)
            fire_wb(g0, 0)

            @pl.when(i < n2 - 1)
            def _():
                fire_gather(g0 + 2, 0)

            drain_gather(1)

            @pl.when(i > 0)
            def _():
                wait_wb(1)

            transpose(1)
            fire_wb(g0 + 1, 1)

            @pl.when(i < n2 - 1)
            def _():
                fire_gather(g0 + 3, 1)

            return carry

        lax.fori_loop(0, n2, body, 0)
        wait_wb(0)
        wait_wb(1)

    return k(weights, idxflat)


def kernel(tokenIndex, weights):
    b, h = tokenIndex.shape
    d = weights.shape[1]
    idxflat = tokenIndex.T.reshape(-1, 128)
    out_phys = _gather_sc(weights, idxflat, h, d)
    return out_phys.transpose(2, 4, 0, 1, 3).reshape(b, h, d)


# speedup vs baseline: 1.6684x; 1.6684x over previous
import functools

import jax
import jax.numpy as jnp
from jax import lax
from jax.experimental import pallas as pl
from jax.experimental.pallas import tpu as pltpu
from jax.experimental.pallas import tpu_sc as plsc

_PITCH = 137


def _gather_sc(weights, idxflat, h_dim, d_dim):
    n_rows, lanes = idxflat.shape
    d_hi = d_dim // 8
    b_hi = n_rows // h_dim
    info = plsc.get_sparse_core_info()
    nc, ns = info.num_cores, info.num_subcores
    nw = nc * ns
    rows_per_w = n_rows // nw
    n2 = rows_per_w // 2

    mesh = plsc.VectorSubcoreMesh(core_axis_name="c", subcore_axis_name="s")

    @functools.partial(
        pl.kernel,
        mesh=mesh,
        compiler_params=pltpu.CompilerParams(
            use_tc_tiling_on_sc=False, needs_layout_passes=False
        ),
        out_type=jax.ShapeDtypeStruct((h_dim, d_hi, b_hi, 8, lanes), jnp.float32),
        scratch_types=[
            pltpu.VMEM((rows_per_w, lanes), jnp.int32),
            pltpu.VMEM((lanes, d_dim), jnp.float32),
            pltpu.VMEM((lanes, d_dim), jnp.float32),
            pltpu.VMEM((d_dim, _PITCH), jnp.float32),
            pltpu.VMEM((d_dim, _PITCH), jnp.float32),
            pltpu.SemaphoreType.DMA,
            pltpu.SemaphoreType.DMA,
            pltpu.SemaphoreType.DMA,
            pltpu.SemaphoreType.DMA,
        ],
    )
    def k(table_hbm, idx_hbm, out_hbm, idx_all, rows0, rows1, out0, out1,
          sg0, sg1, sw0, sw1):
        wid = lax.axis_index("s") * nc + lax.axis_index("c")
        base = wid * rows_per_w
        rows = (rows0, rows1)
        outs = (out0, out1)
        sg = (sg0, sg1)
        sw = (sw0, sw1)

        pltpu.sync_copy(idx_hbm.at[pl.ds(base, rows_per_w)], idx_all)

        iota = lax.iota(jnp.int32, 16)
        ddvecs = [iota + (16 * k) for k in range(d_dim // 16)]

        def fire_gather(g, p):
            pltpu.make_async_copy(
                table_hbm.at[idx_all.at[g]], rows[p], sg[p]
            ).start()

        def drain_gather(p):
            pltpu.make_async_copy(
                table_hbm.at[pl.ds(0, lanes)], rows[p], sg[p]
            ).wait()

        def transpose(p):
            r, o = rows[p], outs[p]
            for bl in range(lanes):
                blvec = jnp.full((16,), bl, jnp.int32)
                for kk in range(d_dim // 16):
                    val = r[bl, pl.ds(16 * kk, 16)]
                    plsc.store_scatter(o, [ddvecs[kk], blvec], val)

        def fire_wb(g, p):
            c = base + g
            h = c // b_hi
            bc = lax.rem(c, b_hi)
            for dr in range(d_hi):
                pltpu.make_async_copy(
                    outs[p].at[pl.ds(8 * dr, 8), pl.ds(0, lanes)],
                    out_hbm.at[h, dr, bc],
                    sw[p],
                ).start()

        def wait_wb(p):
            for dr in range(d_hi):
                pltpu.make_async_copy(
                    outs[p].at[pl.ds(8 * dr, 8), pl.ds(0, lanes)],
                    out_hbm.at[0, dr, 0],
                    sw[p],
                ).wait()

        fire_gather(0, 0)
        fire_gather(1, 1)

        def body(i, carry):
            g0 = 2 * i
            drain_gather(0)

            @pl.when(i > 0)
            def _():
                wait_wb(0)

            transpose(0)
            fire_wb(g0, 0)

            @pl.when(i < n2 - 1)
            def _():
                fire_gather(g0 + 2, 0)

            drain_gather(1)

            @pl.when(i > 0)
            def _():
                wait_wb(1)

            transpose(1)
            fire_wb(g0 + 1, 1)

            @pl.when(i < n2 - 1)
            def _():
                fire_gather(g0 + 3, 1)

            return carry

        lax.fori_loop(0, n2, body, 0)
        wait_wb(0)
        wait_wb(1)

    return k(weights, idxflat)


def kernel(tokenIndex, weights):
    b, h = tokenIndex.shape
    d = weights.shape[1]
    idxflat = tokenIndex.T.reshape(-1, 128)
    out_phys = _gather_sc(weights, idxflat, h, d)
    return out_phys.transpose(2, 4, 0, 1, 3).reshape(b, h, d)


# speedup vs baseline: 2.5353x; 1.5196x over previous
import functools

import jax
import jax.numpy as jnp
from jax import lax
from jax.experimental import pallas as pl
from jax.experimental.pallas import tpu as pltpu
from jax.experimental.pallas import tpu_sc as plsc

_PITCH = 137


def _gather_sc(weights, idxflat, h_dim, d_dim):
    n_rows, lanes = idxflat.shape
    d_hi = d_dim // 8
    b_hi = n_rows // h_dim
    info = plsc.get_sparse_core_info()
    nc, ns = info.num_cores, info.num_subcores
    nw = nc * ns
    rows_per_w = n_rows // nw
    n2 = rows_per_w // 2

    mesh = plsc.VectorSubcoreMesh(core_axis_name="c", subcore_axis_name="s")

    @functools.partial(
        pl.kernel,
        mesh=mesh,
        compiler_params=pltpu.CompilerParams(
            use_tc_tiling_on_sc=False, needs_layout_passes=False
        ),
        out_type=jax.ShapeDtypeStruct((h_dim, d_hi, b_hi, 8, lanes), jnp.float32),
        scratch_types=[
            pltpu.VMEM((rows_per_w, lanes), jnp.int32),
            pltpu.VMEM((lanes, d_dim), jnp.float32),
            pltpu.VMEM((lanes, d_dim), jnp.float32),
            pltpu.VMEM((d_dim, _PITCH), jnp.float32),
            pltpu.VMEM((d_dim, _PITCH), jnp.float32),
            pltpu.SemaphoreType.DMA,
            pltpu.SemaphoreType.DMA,
            pltpu.SemaphoreType.DMA,
            pltpu.SemaphoreType.DMA,
        ],
    )
    def k(table_hbm, idx_hbm, out_hbm, idx_all, rows0, rows1, out0, out1,
          sg0, sg1, sw0, sw1):
        wid = lax.axis_index("s") * nc + lax.axis_index("c")
        base = wid * rows_per_w
        rows = (rows0, rows1)
        outs = (out0, out1)
        sg = (sg0, sg1)
        sw = (sw0, sw1)

        pltpu.sync_copy(idx_hbm.at[pl.ds(base, rows_per_w)], idx_all)

        iota = lax.iota(jnp.int32, 16)
        ddvecs = [iota + (16 * k) for k in range(d_dim // 16)]

        def fire_gather(g, p):
            pltpu.make_async_copy(
                table_hbm.at[idx_all.at[g]], rows[p], sg[p]
            ).start()

        def drain_gather(p):
            pltpu.make_async_copy(
                table_hbm.at[pl.ds(0, lanes)], rows[p], sg[p]
            ).wait()

        def transpose(p):
            r, o = rows[p], outs[p]

            @plsc.parallel_loop(0, lanes, 1, unroll=8)
            def _(bl):
                blvec = jnp.zeros((16,), jnp.int32) + bl
                for kk in range(d_dim // 16):
                    val = r[bl, pl.ds(16 * kk, 16)]
                    plsc.store_scatter(o, [ddvecs[kk], blvec], val)

        def fire_wb(g, p):
            c = base + g
            h = c // b_hi
            bc = lax.rem(c, b_hi)
            for dr in range(d_hi):
                pltpu.make_async_copy(
                    outs[p].at[pl.ds(8 * dr, 8), pl.ds(0, lanes)],
                    out_hbm.at[h, dr, bc],
                    sw[p],
                ).start()

        def wait_wb(p):
            for dr in range(d_hi):
                pltpu.make_async_copy(
                    outs[p].at[pl.ds(8 * dr, 8), pl.ds(0, lanes)],
                    out_hbm.at[0, dr, 0],
                    sw[p],
                ).wait()

        fire_gather(0, 0)
        fire_gather(1, 1)

        def body(i, carry):
            g0 = 2 * i
            drain_gather(0)

            @pl.when(i > 0)
            def _():
                wait_wb(0)

            transpose(0)
            fire_wb(g0, 0)

            @pl.when(i < n2 - 1)
            def _():
                fire_gather(g0 + 2, 0)

            drain_gather(1)

            @pl.when(i > 0)
            def _():
                wait_wb(1)

            transpose(1)
            fire_wb(g0 + 1, 1)

            @pl.when(i < n2 - 1)
            def _():
                fire_gather(g0 + 3, 1)

            return carry

        lax.fori_loop(0, n2, body, 0)
        wait_wb(0)
        wait_wb(1)

    return k(weights, idxflat)


def kernel(tokenIndex, weights):
    b, h = tokenIndex.shape
    d = weights.shape[1]
    idxflat = tokenIndex.T.reshape(-1, 128)
    out_phys = _gather_sc(weights, idxflat, h, d)
    return out_phys.transpose(2, 4, 0, 1, 3).reshape(b, h, d)
